# baseline (speedup 1.0000x reference)
import jax
import jax.numpy as jnp
from jax.experimental import pallas as pl
from jax.experimental.pallas import tpu as pltpu

_TM = 4096


def _mlp_kernel(x_ref, w1_ref, b1_ref, w2_ref, b2_ref, w3_ref, b3_ref,
                out_ref):
    f1 = w1_ref.shape[1]
    f2 = w2_ref.shape[1]
    na = w3_ref.shape[1]

    w1 = jnp.pad(w1_ref[...], ((0, 0), (0, 256 - f1)))
    b1 = jnp.pad(b1_ref[...], ((0, 0), (0, 256 - f1)))
    w2 = jnp.pad(w2_ref[...], ((0, 256 - w2_ref.shape[0]), (0, 256 - f2)))
    b2 = jnp.pad(b2_ref[...], ((0, 0), (0, 256 - f2)))
    w3 = jnp.pad(w3_ref[...], ((0, 256 - w3_ref.shape[0]), (0, 256 - na)))
    b3 = jnp.pad(b3_ref[...], ((0, 0), (0, 256 - na)))

    x = jnp.dot(x_ref[...], w1, preferred_element_type=jnp.float32)
    x = jnp.maximum(x + b1, 0.0)
    x = jnp.dot(x, w2, preferred_element_type=jnp.float32)
    x = jnp.maximum(x + b2, 0.0)
    x = jnp.dot(x, w3, preferred_element_type=jnp.float32)
    x = jnp.tanh(x + b3)
    out_ref[...] = x[:, :na].astype(out_ref.dtype)


def _round_up(x, m):
    return ((x + m - 1) // m) * m


@jax.jit
def _actor_forward(state, w1, b1, w2, b2, w3, b3):
    batch, in_dim = state.shape
    action_dim = w3.shape[1]

    tm = min(_TM, _round_up(batch, 8))
    padded = _round_up(batch, tm)
    if padded != batch:
        state = jnp.pad(state, ((0, padded - batch), (0, 0)))

    grid = (padded // tm,)

    def resident(shape):
        return pl.BlockSpec(shape, lambda i, _s=shape: (0,) * len(_s))

    out = pl.pallas_call(
        _mlp_kernel,
        out_shape=jax.ShapeDtypeStruct((padded, action_dim), jnp.float32),
        grid=grid,
        in_specs=[
            pl.BlockSpec((tm, in_dim), lambda i: (i, 0)),
            resident(w1.shape), resident(b1.shape),
            resident(w2.shape), resident(b2.shape),
            resident(w3.shape), resident(b3.shape),
        ],
        out_specs=pl.BlockSpec((tm, action_dim), lambda i: (i, 0)),
        compiler_params=pltpu.CompilerParams(
            dimension_semantics=("parallel",),
            vmem_limit_bytes=64 * 1024 * 1024,
        ),
    )(state, w1, b1, w2, b2, w3, b3)

    return out[:batch]


def kernel(state, w1, b1, w2, b2, w3, b3):
    return _actor_forward(state, w1, b1, w2, b2, w3, b3)


# speedup vs baseline: 12.7918x; 12.7918x over previous
import jax
import jax.numpy as jnp
from jax.experimental import pallas as pl
from jax.experimental.pallas import tpu as pltpu

_TN = 32768


def _mlp_kernel(x_ref, w1t_ref, b1t_ref, w2t_ref, b2t_ref, w3t_ref, b3t_ref,
                out_ref):
    x = x_ref[...]
    h = jnp.dot(w1t_ref[...], x, preferred_element_type=jnp.float32)
    h = jnp.maximum(h + b1t_ref[...], 0.0)
    h = jnp.dot(w2t_ref[...], h, preferred_element_type=jnp.float32)
    h = jnp.maximum(h + b2t_ref[...], 0.0)
    h = jnp.dot(w3t_ref[...], h, preferred_element_type=jnp.float32)
    out_ref[...] = jnp.tanh(h + b3t_ref[...]).astype(out_ref.dtype)


def _round_up(x, m):
    return ((x + m - 1) // m) * m


@jax.jit
def _actor_forward(state, w1, b1, w2, b2, w3, b3):
    batch, in_dim = state.shape
    action_dim = w3.shape[1]

    xt = state.T
    w1t, w2t, w3t = w1.T, w2.T, w3.T
    b1t, b2t, b3t = b1.T, b2.T, b3.T

    tn = min(_TN, _round_up(batch, 128))
    padded = _round_up(batch, tn)
    if padded != batch:
        xt = jnp.pad(xt, ((0, 0), (0, padded - batch)))

    grid = (padded // tn,)

    def resident(shape):
        return pl.BlockSpec(shape, lambda i, _s=shape: (0,) * len(_s))

    out_t = pl.pallas_call(
        _mlp_kernel,
        out_shape=jax.ShapeDtypeStruct((action_dim, padded), jnp.float32),
        grid=grid,
        in_specs=[
            pl.BlockSpec((in_dim, tn), lambda i: (0, i)),
            resident(w1t.shape), resident(b1t.shape),
            resident(w2t.shape), resident(b2t.shape),
            resident(w3t.shape), resident(b3t.shape),
        ],
        out_specs=pl.BlockSpec((action_dim, tn), lambda i: (0, i)),
        compiler_params=pltpu.CompilerParams(
            dimension_semantics=("parallel",),
            vmem_limit_bytes=64 * 1024 * 1024,
        ),
    )(xt, w1t, b1t, w2t, b2t, w3t, b3t)

    return out_t[:, :batch].T


def kernel(state, w1, b1, w2, b2, w3, b3):
    return _actor_forward(state, w1, b1, w2, b2, w3, b3)


# speedup vs baseline: 14.2775x; 1.1161x over previous
import jax
import jax.numpy as jnp
from jax.experimental import pallas as pl
from jax.experimental.pallas import tpu as pltpu

_TN = 32768


def _mlp_kernel(x_ref, w1_ref, b1_ref, w2t_ref, b2_ref, w3t_ref, b3_ref,
                out_ref):
    x = x_ref[...]
    h = jax.lax.dot_general(w1_ref[...], x, (((0,), (0,)), ((), ())),
                            preferred_element_type=jnp.float32)
    h = jnp.maximum(h + b1_ref[...].T, 0.0)
    h = jnp.dot(w2t_ref[...], h, preferred_element_type=jnp.float32)
    h = jnp.maximum(h + b2_ref[...].T, 0.0)
    h = jnp.dot(w3t_ref[...], h, preferred_element_type=jnp.float32)
    out_ref[...] = jnp.tanh(h + b3_ref[...].T).astype(out_ref.dtype)


def _round_up(x, m):
    return ((x + m - 1) // m) * m


@jax.jit
def _actor_forward(state, w1, b1, w2, b2, w3, b3):
    batch, in_dim = state.shape
    action_dim = w3.shape[1]

    xt = state.T
    w2t, w3t = w2.T, w3.T

    tn = min(_TN, _round_up(batch, 128))
    padded = _round_up(batch, tn)
    if padded != batch:
        xt = jnp.pad(xt, ((0, 0), (0, padded - batch)))

    grid = (padded // tn,)

    def resident(shape):
        return pl.BlockSpec(shape, lambda i, _s=shape: (0,) * len(_s))

    out_t = pl.pallas_call(
        _mlp_kernel,
        out_shape=jax.ShapeDtypeStruct((action_dim, padded), jnp.float32),
        grid=grid,
        in_specs=[
            pl.BlockSpec((in_dim, tn), lambda i: (0, i)),
            resident(w1.shape), resident(b1.shape),
            resident(w2t.shape), resident(b2.shape),
            resident(w3t.shape), resident(b3.shape),
        ],
        out_specs=pl.BlockSpec((action_dim, tn), lambda i: (0, i)),
        compiler_params=pltpu.CompilerParams(
            dimension_semantics=("parallel",),
            vmem_limit_bytes=64 * 1024 * 1024,
        ),
    )(xt, w1, b1, w2t, b2, w3t, b3)

    return out_t[:, :batch].T


def kernel(state, w1, b1, w2, b2, w3, b3):
    return _actor_forward(state, w1, b1, w2, b2, w3, b3)


# speedup vs baseline: 14.6593x; 1.0267x over previous
import jax
import jax.numpy as jnp
from jax.experimental import pallas as pl
from jax.experimental.pallas import tpu as pltpu

_TN = 65536


def _mlp_kernel(x_ref, w1_ref, b1_ref, w2t_ref, b2_ref, w3t_ref, b3_ref,
                out_ref):
    x = x_ref[...]
    h = jax.lax.dot_general(w1_ref[...], x, (((0,), (0,)), ((), ())),
                            preferred_element_type=jnp.float32)
    h = jnp.maximum(h + b1_ref[...].T, 0.0)
    h = jnp.dot(w2t_ref[...], h, preferred_element_type=jnp.float32)
    h = jnp.maximum(h + b2_ref[...].T, 0.0)
    h = jnp.dot(w3t_ref[...], h, preferred_element_type=jnp.float32)
    out_ref[...] = jnp.tanh(h + b3_ref[...].T).astype(out_ref.dtype)


def _round_up(x, m):
    return ((x + m - 1) // m) * m


@jax.jit
def _actor_forward(state, w1, b1, w2, b2, w3, b3):
    batch, in_dim = state.shape
    action_dim = w3.shape[1]

    xt = state.T
    w2t, w3t = w2.T, w3.T

    tn = min(_TN, _round_up(batch, 128))
    padded = _round_up(batch, tn)
    if padded != batch:
        xt = jnp.pad(xt, ((0, 0), (0, padded - batch)))

    grid = (padded // tn,)

    def resident(shape):
        return pl.BlockSpec(shape, lambda i, _s=shape: (0,) * len(_s))

    out_t = pl.pallas_call(
        _mlp_kernel,
        out_shape=jax.ShapeDtypeStruct((action_dim, padded), jnp.float32),
        grid=grid,
        in_specs=[
            pl.BlockSpec((in_dim, tn), lambda i: (0, i)),
            resident(w1.shape), resident(b1.shape),
            resident(w2t.shape), resident(b2.shape),
            resident(w3t.shape), resident(b3.shape),
        ],
        out_specs=pl.BlockSpec((action_dim, tn), lambda i: (0, i)),
        compiler_params=pltpu.CompilerParams(
            dimension_semantics=("parallel",),
            vmem_limit_bytes=64 * 1024 * 1024,
        ),
    )(xt, w1, b1, w2t, b2, w3t, b3)

    return out_t[:, :batch].T


def kernel(state, w1, b1, w2, b2, w3, b3):
    return _actor_forward(state, w1, b1, w2, b2, w3, b3)
